# Initial kernel scaffold; baseline (speedup 1.0000x reference)
import jax
import jax.numpy as jnp
from jax.experimental import pallas as pl


def kernel(x, x_fields, x_positions, token_table, field_table, pos_table):
    raise NotImplementedError("write your pallas kernel here")


# speedup vs baseline: 3.3490x; 3.3490x over previous
import functools

import jax
import jax.numpy as jnp
from jax import lax
from jax.experimental import pallas as pl
from jax.experimental.pallas import tpu as pltpu
from jax.experimental.pallas import tpu_sc as plsc

VOCAB = 1000000
NB_FIELDS = 26
SEQ_LEN = 200
EMBED_DIM = 64
BATCH = 4096

N_ROWS = BATCH * SEQ_LEN
NC, NS, L = 2, 16, 16
NW = NC * NS
ROWS_PER_W = N_ROWS // NW
CHUNK = 512
SUB = 128
N_CHUNKS = ROWS_PER_W // CHUNK

_GATHER_DN = lax.GatherDimensionNumbers(
    offset_dims=(), collapsed_slice_dims=(0,), start_index_map=(0,))


def _dyn_gather(vec, sel):
    return lax.gather(vec, sel[:, None], _GATHER_DN, (1,),
                      mode=lax.GatherScatterMode.PROMISE_IN_BOUNDS)


def _body(tok_idx_hbm, fidx_hbm, pidx_hbm, table_hbm, fld_hbm, pos_hbm,
          out_hbm, tok_idx_v, fidx_v, pidx_v, rows_v, fld_v, pos_v, sem):
    wid = lax.axis_index("s") * NC + lax.axis_index("c")
    base = wid * ROWS_PER_W

    pltpu.sync_copy(fld_hbm, fld_v)
    pltpu.sync_copy(pos_hbm, pos_v)

    iota = lax.iota(jnp.int32, L)

    def chunk_body(g, carry):
        row0 = base + g * CHUNK
        pltpu.sync_copy(tok_idx_hbm.at[pl.ds(row0, CHUNK)], tok_idx_v)
        pltpu.sync_copy(fidx_hbm.at[pl.ds(row0, CHUNK)], fidx_v)
        pltpu.sync_copy(pidx_hbm.at[pl.ds(row0, CHUNK)], pidx_v)

        for k in range(CHUNK // SUB):
            pltpu.async_copy(table_hbm.at[tok_idx_v.at[pl.ds(k * SUB, SUB)]],
                             rows_v.at[pl.ds(k * SUB, SUB), :], sem)
        for k in range(CHUNK // SUB):
            pltpu.make_async_copy(
                table_hbm.at[tok_idx_v.at[pl.ds(k * SUB, SUB)]],
                rows_v.at[pl.ds(k * SUB, SUB), :], sem).wait()

        def block_body(rb, carry2):
            r0 = rb * L
            f16 = fidx_v[pl.ds(r0, L)]
            p16 = pidx_v[pl.ds(r0, L)]
            for j in range(L):
                sel = jnp.full((L,), j, dtype=jnp.int32)
                fj = _dyn_gather(f16, sel)
                pj = _dyn_gather(p16, sel)
                fo = fj * EMBED_DIM + iota
                po = pj * EMBED_DIM + iota
                for c in range(EMBED_DIM // L):
                    fv = plsc.load_gather(fld_v, [fo + (c * L)])
                    pv = plsc.load_gather(pos_v, [po + (c * L)])
                    plsc.addupdate(rows_v.at[r0 + j, pl.ds(c * L, L)],
                                   fv + pv)
            return carry2

        lax.fori_loop(0, CHUNK // L, block_body, 0)

        pltpu.sync_copy(rows_v, out_hbm.at[pl.ds(row0, CHUNK), :])
        return carry

    lax.fori_loop(0, N_CHUNKS, chunk_body, 0)


@jax.jit
def _run(tok_idx, fidx, pidx, table, fld, pos):
    mesh = plsc.VectorSubcoreMesh(core_axis_name="c", subcore_axis_name="s")
    f = pl.kernel(
        _body,
        mesh=mesh,
        out_type=jax.ShapeDtypeStruct((N_ROWS, EMBED_DIM), jnp.float32),
        scratch_types=[
            pltpu.VMEM((CHUNK,), jnp.int32),
            pltpu.VMEM((CHUNK,), jnp.int32),
            pltpu.VMEM((CHUNK,), jnp.int32),
            pltpu.VMEM((CHUNK, EMBED_DIM), jnp.float32),
            pltpu.VMEM((NB_FIELDS * EMBED_DIM,), jnp.float32),
            pltpu.VMEM((SEQ_LEN * EMBED_DIM,), jnp.float32),
            pltpu.SemaphoreType.DMA,
        ],
        compiler_params=pltpu.CompilerParams(
            needs_layout_passes=False, use_tc_tiling_on_sc=False),
    )
    return f(tok_idx, fidx, pidx, table, fld, pos)


def kernel(x, x_fields, x_positions, token_table, field_table, pos_table):
    tok_idx = x.astype(jnp.int32).reshape(-1)
    fidx = x_fields.astype(jnp.int32).reshape(-1)
    pidx = x_positions.astype(jnp.int32).reshape(-1)
    out = _run(tok_idx, fidx, pidx, token_table,
               field_table.reshape(-1), pos_table.reshape(-1))
    return out.reshape(BATCH, SEQ_LEN, EMBED_DIM)


# speedup vs baseline: 3.7472x; 1.1189x over previous
import functools

import jax
import jax.numpy as jnp
from jax import lax
from jax.experimental import pallas as pl
from jax.experimental.pallas import tpu as pltpu
from jax.experimental.pallas import tpu_sc as plsc

VOCAB = 1000000
NB_FIELDS = 26
SEQ_LEN = 200
EMBED_DIM = 64
BATCH = 4096

N_ROWS = BATCH * SEQ_LEN
NC, NS, L = 2, 16, 16
NW = NC * NS
ROWS_PER_W = N_ROWS // NW
CHUNK = 512
SUB = 128
N_CHUNKS = ROWS_PER_W // CHUNK
NB = 3
N_GROUPS = (N_CHUNKS + NB - 1) // NB

_GATHER_DN = lax.GatherDimensionNumbers(
    offset_dims=(), collapsed_slice_dims=(0,), start_index_map=(0,))


def _dyn_gather(vec, sel):
    return lax.gather(vec, sel[:, None], _GATHER_DN, (1,),
                      mode=lax.GatherScatterMode.PROMISE_IN_BOUNDS)


def _body(tok_idx_hbm, fidx_hbm, pidx_hbm, table_hbm, fld_hbm, pos_hbm,
          out_hbm, tik_v, fid_v, pid_v, rows_v, fld_v, pos_v,
          g0, g1, g2, o0, o1, o2, isem):
    gsems = (g0, g1, g2)
    osems = (o0, o1, o2)
    wid = lax.axis_index("s") * NC + lax.axis_index("c")
    base = wid * ROWS_PER_W

    pltpu.sync_copy(fld_hbm, fld_v)
    pltpu.sync_copy(pos_hbm, pos_v)

    iota = lax.iota(jnp.int32, L)

    def copy_idx(g, u):
        row0 = base + g * CHUNK
        pltpu.async_copy(tok_idx_hbm.at[pl.ds(row0, CHUNK)], tik_v.at[u],
                         isem)
        pltpu.async_copy(fidx_hbm.at[pl.ds(row0, CHUNK)], fid_v.at[u], isem)
        pltpu.async_copy(pidx_hbm.at[pl.ds(row0, CHUNK)], pid_v.at[u], isem)
        pltpu.make_async_copy(tok_idx_hbm.at[pl.ds(row0, CHUNK)],
                              tik_v.at[u], isem).wait()
        pltpu.make_async_copy(fidx_hbm.at[pl.ds(row0, CHUNK)],
                              fid_v.at[u], isem).wait()
        pltpu.make_async_copy(pidx_hbm.at[pl.ds(row0, CHUNK)],
                              pid_v.at[u], isem).wait()

    def gather_descs(u, sem):
        return [pltpu.make_async_copy(
            table_hbm.at[tik_v.at[u, pl.ds(k * SUB, SUB)]],
            rows_v.at[u, pl.ds(k * SUB, SUB), :], sem)
            for k in range(CHUNK // SUB)]

    def fire_gather(u, sem):
        for d in gather_descs(u, sem):
            d.start()

    def wait_gather(u, sem):
        for d in gather_descs(u, sem):
            d.wait()

    def out_desc(g, u, sem):
        row0 = base + g * CHUNK
        return pltpu.make_async_copy(
            rows_v.at[u], out_hbm.at[pl.ds(row0, CHUNK), :], sem)

    def compute(u):
        def block_body(rb, carry):
            r0 = rb * L
            f16 = fid_v[u, pl.ds(r0, L)]
            p16 = pid_v[u, pl.ds(r0, L)]
            for j in range(L):
                sel = jnp.full((L,), j, dtype=jnp.int32)
                fo = _dyn_gather(f16, sel) * EMBED_DIM + iota
                po = _dyn_gather(p16, sel) * EMBED_DIM + iota
                for c in range(EMBED_DIM // L):
                    fv = plsc.load_gather(fld_v, [fo + (c * L)])
                    pv = plsc.load_gather(pos_v, [po + (c * L)])
                    plsc.addupdate(rows_v.at[u, r0 + j, pl.ds(c * L, L)],
                                   fv + pv)
            return carry

        lax.fori_loop(0, CHUNK // L, block_body, 0)

    copy_idx(0, 0)
    fire_gather(0, gsems[0])

    def group_body(go, carry):
        for u in range(NB):
            g = go * NB + u
            bn = (u + 1) % NB

            @pl.when(g < N_CHUNKS - 1)
            def _prefetch():
                copy_idx(g + 1, bn)

                @pl.when(g >= NB - 1)
                def _reuse_wait():
                    out_desc(g - (NB - 1), bn, osems[bn]).wait()

                fire_gather(bn, gsems[bn])

            @pl.when(g < N_CHUNKS)
            def _work():
                wait_gather(u, gsems[u])
                compute(u)
                out_desc(g, u, osems[u]).start()
        return carry

    lax.fori_loop(0, N_GROUPS, group_body, 0)

    for g in range(N_CHUNKS - NB, N_CHUNKS):
        u = g % NB
        out_desc(g, u, osems[u]).wait()


@jax.jit
def _run(tok_idx, fidx, pidx, table, fld, pos):
    mesh = plsc.VectorSubcoreMesh(core_axis_name="c", subcore_axis_name="s")
    f = pl.kernel(
        _body,
        mesh=mesh,
        out_type=jax.ShapeDtypeStruct((N_ROWS, EMBED_DIM), jnp.float32),
        scratch_types=[
            pltpu.VMEM((NB, CHUNK), jnp.int32),
            pltpu.VMEM((NB, CHUNK), jnp.int32),
            pltpu.VMEM((NB, CHUNK), jnp.int32),
            pltpu.VMEM((NB, CHUNK, EMBED_DIM), jnp.float32),
            pltpu.VMEM((NB_FIELDS * EMBED_DIM,), jnp.float32),
            pltpu.VMEM((SEQ_LEN * EMBED_DIM,), jnp.float32),
            pltpu.SemaphoreType.DMA,
            pltpu.SemaphoreType.DMA,
            pltpu.SemaphoreType.DMA,
            pltpu.SemaphoreType.DMA,
            pltpu.SemaphoreType.DMA,
            pltpu.SemaphoreType.DMA,
            pltpu.SemaphoreType.DMA,
        ],
        compiler_params=pltpu.CompilerParams(
            needs_layout_passes=False, use_tc_tiling_on_sc=False),
    )
    return f(tok_idx, fidx, pidx, table, fld, pos)


def kernel(x, x_fields, x_positions, token_table, field_table, pos_table):
    tok_idx = x.astype(jnp.int32).reshape(-1)
    fidx = x_fields.astype(jnp.int32).reshape(-1)
    pidx = x_positions.astype(jnp.int32).reshape(-1)
    out = _run(tok_idx, fidx, pidx, token_table,
               field_table.reshape(-1), pos_table.reshape(-1))
    return out.reshape(BATCH, SEQ_LEN, EMBED_DIM)


# speedup vs baseline: 4.4962x; 1.1999x over previous
import functools

import jax
import jax.numpy as jnp
from jax import lax
from jax.experimental import pallas as pl
from jax.experimental.pallas import tpu as pltpu
from jax.experimental.pallas import tpu_sc as plsc

VOCAB = 1000000
NB_FIELDS = 26
SEQ_LEN = 200
EMBED_DIM = 64
BATCH = 4096

N_ROWS = BATCH * SEQ_LEN
NC, NS, L = 2, 16, 16
NW = NC * NS
GRP = 128
N_GROUPS_TOT = N_ROWS // GRP
GROUPS_PER_W = N_GROUPS_TOT // NW
GPC = 2
CHUNK = GPC * GRP
N_CHUNKS = GROUPS_PER_W // GPC
NB = 3
N_GROUPS_LOOP = (N_CHUNKS + NB - 1) // NB
PITCH = GRP + 1

_GATHER_DN = lax.GatherDimensionNumbers(
    offset_dims=(), collapsed_slice_dims=(0,), start_index_map=(0,))


def _dyn_gather(vec, sel):
    return lax.gather(vec, sel[:, None], _GATHER_DN, (1,),
                      mode=lax.GatherScatterMode.PROMISE_IN_BOUNDS)


def _body(tok_idx_hbm, fidx_hbm, pidx_hbm, table_hbm, fld_hbm, pos_hbm,
          out_hbm, tik_v, fid_v, pid_v, rows_v, tp_v, fld_v, pos_v,
          g0, g1, g2, o0, o1, o2, isem):
    gsems = (g0, g1, g2)
    osems = (o0, o1, o2)
    wid = lax.axis_index("s") * NC + lax.axis_index("c")
    gbase = wid * GROUPS_PER_W

    pltpu.sync_copy(fld_hbm, fld_v)
    pltpu.sync_copy(pos_hbm, pos_v)

    iota = lax.iota(jnp.int32, L)
    etr_c = [(16 * c + iota) // 8 for c in range(EMBED_DIM // L)]
    ees_c = [(16 * c + iota) % 8 for c in range(EMBED_DIM // L)]

    def copy_idx(g, u):
        row0 = (gbase + g * GPC) * GRP
        for src, dst in ((tok_idx_hbm, tik_v), (fidx_hbm, fid_v),
                         (pidx_hbm, pid_v)):
            pltpu.async_copy(src.at[pl.ds(row0, CHUNK)], dst.at[u], isem)
        for src, dst in ((tok_idx_hbm, tik_v), (fidx_hbm, fid_v),
                         (pidx_hbm, pid_v)):
            pltpu.make_async_copy(src.at[pl.ds(row0, CHUNK)], dst.at[u],
                                  isem).wait()

    def gather_descs(u, sem):
        return [pltpu.make_async_copy(
            table_hbm.at[tik_v.at[u, pl.ds(k * GRP, GRP)]],
            rows_v.at[u, pl.ds(k * GRP, GRP), :], sem)
            for k in range(GPC)]

    def out_descs(g, u, sem):
        descs = []
        for k in range(GPC):
            grp = gbase + g * GPC + k
            s = ((grp // 256) * 8) | (grp % 8)
            tc = (grp // 8) % 32
            descs.append(pltpu.make_async_copy(
                tp_v.at[u, k, :, :, pl.ds(0, GRP)],
                out_hbm.at[s, :, tc, :, :], sem))
        return descs

    def compute(g, u):
        for k in range(GPC):

            def bg_body(bg, carry):
                f16 = fid_v[u, pl.ds(k * GRP + bg * L, L)]
                p16 = pid_v[u, pl.ds(k * GRP + bg * L, L)]
                for j in range(L):
                    t = bg * L + j
                    row = k * GRP + t
                    sel = jnp.full((L,), j, jnp.int32)
                    foff = _dyn_gather(f16, sel) * EMBED_DIM
                    poff = _dyn_gather(p16, sel) * EMBED_DIM
                    bl16 = jnp.broadcast_to(t, (L,)).astype(jnp.int32)
                    for c in range(EMBED_DIM // L):
                        tok = rows_v[u, row, pl.ds(16 * c, L)]
                        fv = plsc.load_gather(
                            fld_v, [foff + (16 * c) + iota])
                        pv = plsc.load_gather(
                            pos_v, [poff + (16 * c) + iota])
                        plsc.store_scatter(
                            tp_v.at[u, k], [etr_c[c], ees_c[c], bl16],
                            tok + fv + pv)
                return carry

            lax.fori_loop(0, GRP // L, bg_body, 0)

    copy_idx(0, 0)
    for d in gather_descs(0, gsems[0]):
        d.start()

    def group_body(go, carry):
        for u in range(NB):
            g = go * NB + u
            bn = (u + 1) % NB

            @pl.when(g < N_CHUNKS - 1)
            def _prefetch():
                copy_idx(g + 1, bn)
                for d in gather_descs(bn, gsems[bn]):
                    d.start()

            @pl.when(g < N_CHUNKS)
            def _work():
                for d in gather_descs(u, gsems[u]):
                    d.wait()

                @pl.when(g >= NB)
                def _reuse_wait():
                    for d in out_descs(g - NB, u, osems[u]):
                        d.wait()

                compute(g, u)
                for d in out_descs(g, u, osems[u]):
                    d.start()
        return carry

    lax.fori_loop(0, N_GROUPS_LOOP, group_body, 0)

    for g in range(N_CHUNKS - NB, N_CHUNKS):
        for d in out_descs(g, g % NB, osems[g % NB]):
            d.wait()


@jax.jit
def _run(tok_idx, fidx, pidx, table, fld, pos):
    mesh = plsc.VectorSubcoreMesh(core_axis_name="c", subcore_axis_name="s")
    f = pl.kernel(
        _body,
        mesh=mesh,
        out_type=jax.ShapeDtypeStruct(
            (SEQ_LEN, EMBED_DIM // 8, BATCH // GRP, 8, GRP), jnp.float32),
        scratch_types=[
            pltpu.VMEM((NB, CHUNK), jnp.int32),
            pltpu.VMEM((NB, CHUNK), jnp.int32),
            pltpu.VMEM((NB, CHUNK), jnp.int32),
            pltpu.VMEM((NB, CHUNK, EMBED_DIM), jnp.float32),
            pltpu.VMEM((NB, GPC, 8, 8, PITCH), jnp.float32),
            pltpu.VMEM((NB_FIELDS * EMBED_DIM,), jnp.float32),
            pltpu.VMEM((SEQ_LEN * EMBED_DIM,), jnp.float32),
            pltpu.SemaphoreType.DMA,
            pltpu.SemaphoreType.DMA,
            pltpu.SemaphoreType.DMA,
            pltpu.SemaphoreType.DMA,
            pltpu.SemaphoreType.DMA,
            pltpu.SemaphoreType.DMA,
            pltpu.SemaphoreType.DMA,
        ],
        compiler_params=pltpu.CompilerParams(
            needs_layout_passes=False, use_tc_tiling_on_sc=False),
    )
    return f(tok_idx, fidx, pidx, table, fld, pos)


def _tile_flatten(a):
    return (a.astype(jnp.int32).T
            .reshape(SEQ_LEN // 8, 8, BATCH // GRP, GRP)
            .transpose(0, 2, 1, 3).reshape(-1))


def kernel(x, x_fields, x_positions, token_table, field_table, pos_table):
    tok_idx = _tile_flatten(x)
    fidx = _tile_flatten(x_fields)
    pidx = _tile_flatten(x_positions)
    out5 = _run(tok_idx, fidx, pidx, token_table,
                field_table.reshape(-1), pos_table.reshape(-1))
    return (out5.transpose(2, 4, 0, 1, 3)
            .reshape(BATCH, SEQ_LEN, EMBED_DIM))


# speedup vs baseline: 4.7202x; 1.0498x over previous
import functools

import jax
import jax.numpy as jnp
from jax import lax
from jax.experimental import pallas as pl
from jax.experimental.pallas import tpu as pltpu
from jax.experimental.pallas import tpu_sc as plsc

VOCAB = 1000000
NB_FIELDS = 26
SEQ_LEN = 200
EMBED_DIM = 64
BATCH = 4096

N_ROWS = BATCH * SEQ_LEN
NC, NS, L = 2, 16, 16
NW = NC * NS
GRP = 128
N_GROUPS_TOT = N_ROWS // GRP
GROUPS_PER_W = N_GROUPS_TOT // NW
GPC = 2
CHUNK = GPC * GRP
N_CHUNKS = GROUPS_PER_W // GPC
NB = 3
N_GROUPS_LOOP = (N_CHUNKS + NB - 1) // NB
PITCH = GRP + 1

_GATHER_DN = lax.GatherDimensionNumbers(
    offset_dims=(), collapsed_slice_dims=(0,), start_index_map=(0,))


def _dyn_gather(vec, sel):
    return lax.gather(vec, sel[:, None], _GATHER_DN, (1,),
                      mode=lax.GatherScatterMode.PROMISE_IN_BOUNDS)


def _body(tok_idx_hbm, fidx_hbm, pidx_hbm, table_hbm, fld_hbm, pos_hbm,
          out_hbm, tik_v, fid_v, pid_v, rows_v, tp_v, fld_v, pos_v,
          g0, g1, g2, o0, o1, o2, isem):
    gsems = (g0, g1, g2)
    osems = (o0, o1, o2)
    wid = lax.axis_index("s") * NC + lax.axis_index("c")
    gbase = wid * GROUPS_PER_W

    pltpu.sync_copy(fld_hbm, fld_v)
    pltpu.sync_copy(pos_hbm, pos_v)

    iota = lax.iota(jnp.int32, L)
    etr_c = [(16 * c + iota) // 8 for c in range(EMBED_DIM // L)]
    ees_c = [(16 * c + iota) % 8 for c in range(EMBED_DIM // L)]

    def idx_descs(g, u):
        row0 = (gbase + g * GPC) * GRP
        return [pltpu.make_async_copy(src.at[pl.ds(row0, CHUNK)],
                                      dst.at[u], isem)
                for src, dst in ((tok_idx_hbm, tik_v), (fidx_hbm, fid_v),
                                 (pidx_hbm, pid_v))]

    def gather_descs(u, sem):
        return [pltpu.make_async_copy(
            table_hbm.at[tik_v.at[u]],
            rows_v.at[u], sem)]

    def out_descs(g, u, sem):
        descs = []
        for k in range(GPC):
            grp = gbase + g * GPC + k
            s = ((grp // 256) * 8) | (grp % 8)
            tc = (grp // 8) % 32
            descs.append(pltpu.make_async_copy(
                tp_v.at[u, k, :, :, pl.ds(0, GRP)],
                out_hbm.at[s, :, tc, :, :], sem))
        return descs

    def compute(g, u):
        for k in range(GPC):

            def bg_body(bg, carry):
                f16 = fid_v[u, pl.ds(k * GRP + bg * L, L)]
                p16 = pid_v[u, pl.ds(k * GRP + bg * L, L)]
                for j in range(L):
                    t = bg * L + j
                    row = k * GRP + t
                    fo = f16[j] * EMBED_DIM
                    po = p16[j] * EMBED_DIM
                    bl16 = jnp.broadcast_to(t, (L,)).astype(jnp.int32)
                    for c in range(EMBED_DIM // L):
                        tok = rows_v[u, row, pl.ds(16 * c, L)]
                        fv = fld_v[pl.ds(fo + 16 * c, L)]
                        pv = pos_v[pl.ds(po + 16 * c, L)]
                        plsc.store_scatter(
                            tp_v.at[u, k], [etr_c[c], ees_c[c], bl16],
                            tok + fv + pv)
                return carry

            lax.fori_loop(0, GRP // L, bg_body, 0)

    for d in idx_descs(0, 0):
        d.start()
    for d in idx_descs(0, 0):
        d.wait()
    for d in gather_descs(0, gsems[0]):
        d.start()
    for d in idx_descs(1, 1):
        d.start()
    for d in idx_descs(1, 1):
        d.wait()
    for d in gather_descs(1, gsems[1]):
        d.start()
    for d in idx_descs(2, 2):
        d.start()

    def group_body(go, carry):
        for u in range(NB):
            g = go * NB + u
            un = (u + 2) % NB

            @pl.when(g < N_CHUNKS)
            def _work():
                for d in gather_descs(u, gsems[u]):
                    d.wait()

                @pl.when(g >= NB)
                def _reuse_wait():
                    for d in out_descs(g - NB, u, osems[u]):
                        d.wait()

                compute(g, u)
                for d in out_descs(g, u, osems[u]):
                    d.start()

                @pl.when(g < N_CHUNKS - 2)
                def _prefetch():
                    for d in idx_descs(g + 2, un):
                        d.wait()
                    for d in gather_descs(un, gsems[un]):
                        d.start()

                    @pl.when(g < N_CHUNKS - 3)
                    def _idx_ahead():
                        for d in idx_descs(g + 3, u):
                            d.start()
        return carry

    lax.fori_loop(0, N_GROUPS_LOOP, group_body, 0)

    for g in range(N_CHUNKS - NB, N_CHUNKS):
        for d in out_descs(g, g % NB, osems[g % NB]):
            d.wait()


@jax.jit
def _run(tok_idx, fidx, pidx, table, fld, pos):
    mesh = plsc.VectorSubcoreMesh(core_axis_name="c", subcore_axis_name="s")
    f = pl.kernel(
        _body,
        mesh=mesh,
        out_type=jax.ShapeDtypeStruct(
            (SEQ_LEN, EMBED_DIM // 8, BATCH // GRP, 8, GRP), jnp.float32),
        scratch_types=[
            pltpu.VMEM((NB, CHUNK), jnp.int32),
            pltpu.VMEM((NB, CHUNK), jnp.int32),
            pltpu.VMEM((NB, CHUNK), jnp.int32),
            pltpu.VMEM((NB, CHUNK, EMBED_DIM), jnp.float32),
            pltpu.VMEM((NB, GPC, 8, 8, PITCH), jnp.float32),
            pltpu.VMEM((NB_FIELDS * EMBED_DIM,), jnp.float32),
            pltpu.VMEM((SEQ_LEN * EMBED_DIM,), jnp.float32),
            pltpu.SemaphoreType.DMA,
            pltpu.SemaphoreType.DMA,
            pltpu.SemaphoreType.DMA,
            pltpu.SemaphoreType.DMA,
            pltpu.SemaphoreType.DMA,
            pltpu.SemaphoreType.DMA,
            pltpu.SemaphoreType.DMA,
        ],
        compiler_params=pltpu.CompilerParams(
            needs_layout_passes=False, use_tc_tiling_on_sc=False),
    )
    return f(tok_idx, fidx, pidx, table, fld, pos)


def _tile_flatten(a):
    return (a.astype(jnp.int32).T
            .reshape(SEQ_LEN // 8, 8, BATCH // GRP, GRP)
            .transpose(0, 2, 1, 3).reshape(-1))


def kernel(x, x_fields, x_positions, token_table, field_table, pos_table):
    tok_idx = _tile_flatten(x)
    fidx = _tile_flatten(x_fields)
    pidx = _tile_flatten(x_positions)
    out5 = _run(tok_idx, fidx, pidx, token_table,
                field_table.reshape(-1), pos_table.reshape(-1))
    return (out5.transpose(2, 4, 0, 1, 3)
            .reshape(BATCH, SEQ_LEN, EMBED_DIM))


# speedup vs baseline: 5.3336x; 1.1300x over previous
import functools

import jax
import jax.numpy as jnp
from jax import lax
from jax.experimental import pallas as pl
from jax.experimental.pallas import tpu as pltpu
from jax.experimental.pallas import tpu_sc as plsc

VOCAB = 1000000
NB_FIELDS = 26
SEQ_LEN = 200
EMBED_DIM = 64
BATCH = 4096

N_ROWS = BATCH * SEQ_LEN
NC, NS, L = 2, 16, 16
NW = NC * NS
GRP = 128
N_GROUPS_TOT = N_ROWS // GRP
GROUPS_PER_W = N_GROUPS_TOT // NW
CHUNK = GRP
N_CHUNKS = GROUPS_PER_W
NB = 3
N_GROUPS_LOOP = (N_CHUNKS + NB - 1) // NB
PITCH = GRP + 1
NFP = NB_FIELDS * SEQ_LEN
FP_PER_SUB = 336
FP_BATCH = 48
NFP_ALLOC = NW // NC * FP_PER_SUB


def _body(tok_idx_hbm, fidx_hbm, pidx_hbm, table_hbm, fld_hbm, pos_hbm,
          out_hbm, tik_v, fid_v, pid_v, fpi_v, rows_v, fpr_v, tp_v,
          fld_v, pos_v, stage_v, fp_sh,
          g0, g1, g2, f0, f1, f2, o0, o1, o2, isem):
    gsems = (g0, g1, g2)
    fsems = (f0, f1, f2)
    osems = (o0, o1, o2)
    cid = lax.axis_index("c")
    sid = lax.axis_index("s")
    wid = sid * NC + cid
    gbase = wid * GROUPS_PER_W

    pltpu.sync_copy(fld_hbm, fld_v.at[pl.ds(0, NB_FIELDS * EMBED_DIM)])
    pltpu.sync_copy(pos_hbm, pos_v)

    iota = lax.iota(jnp.int32, L)
    etr_c = [(16 * c + iota) // 8 for c in range(EMBED_DIM // L)]
    ees_c = [(16 * c + iota) % 8 for c in range(EMBED_DIM // L)]

    def fp_batch(b, carry):
        r0 = sid * FP_PER_SUB + b * FP_BATCH

        def one_row(i, carry2):
            r = r0 + i
            f = r // SEQ_LEN
            p = r % SEQ_LEN
            for c in range(EMBED_DIM // L):
                stage_v[i, pl.ds(16 * c, L)] = (
                    fld_v[pl.ds(f * EMBED_DIM + 16 * c, L)]
                    + pos_v[pl.ds(p * EMBED_DIM + 16 * c, L)])
            return carry2

        lax.fori_loop(0, FP_BATCH, one_row, 0)
        pltpu.sync_copy(stage_v, fp_sh.at[pl.ds(r0, FP_BATCH), :])
        return carry

    lax.fori_loop(0, FP_PER_SUB // FP_BATCH, fp_batch, 0)
    plsc.subcore_barrier()

    def idx_descs(g, u):
        row0 = (gbase + g) * GRP
        return [pltpu.make_async_copy(src.at[pl.ds(row0, CHUNK)],
                                      dst.at[u], isem)
                for src, dst in ((tok_idx_hbm, tik_v), (fidx_hbm, fid_v),
                                 (pidx_hbm, pid_v))]

    def fp_indices(u):
        def one(bq, carry):
            f16 = fid_v[u, pl.ds(bq * L, L)]
            p16 = pid_v[u, pl.ds(bq * L, L)]
            fpi_v[u, pl.ds(bq * L, L)] = f16 * SEQ_LEN + p16
            return carry

        lax.fori_loop(0, GRP // L, one, 0)

    def gather_descs(u):
        return [
            pltpu.make_async_copy(table_hbm.at[tik_v.at[u]],
                                  rows_v.at[u], gsems[u]),
            pltpu.make_async_copy(fp_sh.at[fpi_v.at[u]],
                                  fpr_v.at[u], fsems[u]),
        ]

    def out_desc(g, u):
        grp = gbase + g
        s = ((grp // 256) * 8) | (grp % 8)
        tc = (grp // 8) % 32
        return pltpu.make_async_copy(
            tp_v.at[u, :, :, pl.ds(0, GRP)],
            out_hbm.at[s, :, tc, :, :], osems[u])

    def compute(u):
        def bg_body(bg, carry):
            for j in range(L):
                t = bg * L + j
                bl16 = jnp.broadcast_to(t, (L,)).astype(jnp.int32)
                for c in range(EMBED_DIM // L):
                    tok = rows_v[u, t, pl.ds(16 * c, L)]
                    fp = fpr_v[u, t, pl.ds(16 * c, L)]
                    plsc.store_scatter(
                        tp_v.at[u], [etr_c[c], ees_c[c], bl16],
                        tok + fp)
            return carry

        lax.fori_loop(0, GRP // L, bg_body, 0)

    for g in (0, 1):
        for d in idx_descs(g, g):
            d.start()
        for d in idx_descs(g, g):
            d.wait()
        fp_indices(g)
        for d in gather_descs(g):
            d.start()
    for d in idx_descs(2, 2):
        d.start()

    def group_body(go, carry):
        for u in range(NB):
            g = go * NB + u
            un = (u + 2) % NB

            @pl.when(g < N_CHUNKS)
            def _work():
                for d in gather_descs(u):
                    d.wait()

                @pl.when(g >= NB)
                def _reuse_wait():
                    out_desc(g - NB, u).wait()

                compute(u)
                out_desc(g, u).start()

                @pl.when(g < N_CHUNKS - 2)
                def _prefetch():
                    for d in idx_descs(g + 2, un):
                        d.wait()
                    fp_indices(un)
                    for d in gather_descs(un):
                        d.start()

                    @pl.when(g < N_CHUNKS - 3)
                    def _idx_ahead():
                        for d in idx_descs(g + 3, u):
                            d.start()
        return carry

    lax.fori_loop(0, N_GROUPS_LOOP, group_body, 0)

    for g in range(N_CHUNKS - NB, N_CHUNKS):
        out_desc(g, g % NB).wait()


@jax.jit
def _run(tok_idx, fidx, pidx, table, fld, pos):
    mesh = plsc.VectorSubcoreMesh(core_axis_name="c", subcore_axis_name="s")
    f = pl.kernel(
        _body,
        mesh=mesh,
        out_type=jax.ShapeDtypeStruct(
            (SEQ_LEN, EMBED_DIM // 8, BATCH // GRP, 8, GRP), jnp.float32),
        scratch_types=[
            pltpu.VMEM((NB, CHUNK), jnp.int32),
            pltpu.VMEM((NB, CHUNK), jnp.int32),
            pltpu.VMEM((NB, CHUNK), jnp.int32),
            pltpu.VMEM((NB, CHUNK), jnp.int32),
            pltpu.VMEM((NB, CHUNK, EMBED_DIM), jnp.float32),
            pltpu.VMEM((NB, CHUNK, EMBED_DIM), jnp.float32),
            pltpu.VMEM((NB, 8, 8, PITCH), jnp.float32),
            pltpu.VMEM(((NB_FIELDS + 1) * EMBED_DIM,), jnp.float32),
            pltpu.VMEM((SEQ_LEN * EMBED_DIM,), jnp.float32),
            pltpu.VMEM((FP_BATCH, EMBED_DIM), jnp.float32),
            pltpu.VMEM_SHARED((NFP_ALLOC, EMBED_DIM), jnp.float32),
            pltpu.SemaphoreType.DMA,
            pltpu.SemaphoreType.DMA,
            pltpu.SemaphoreType.DMA,
            pltpu.SemaphoreType.DMA,
            pltpu.SemaphoreType.DMA,
            pltpu.SemaphoreType.DMA,
            pltpu.SemaphoreType.DMA,
            pltpu.SemaphoreType.DMA,
            pltpu.SemaphoreType.DMA,
            pltpu.SemaphoreType.DMA,
        ],
        compiler_params=pltpu.CompilerParams(
            needs_layout_passes=False, use_tc_tiling_on_sc=False),
    )
    return f(tok_idx, fidx, pidx, table, fld, pos)


def _tile_flatten(a):
    return (a.astype(jnp.int32).T
            .reshape(SEQ_LEN // 8, 8, BATCH // GRP, GRP)
            .transpose(0, 2, 1, 3).reshape(-1))


def kernel(x, x_fields, x_positions, token_table, field_table, pos_table):
    tok_idx = _tile_flatten(x)
    fidx = _tile_flatten(x_fields)
    pidx = _tile_flatten(x_positions)
    out5 = _run(tok_idx, fidx, pidx, token_table,
                field_table.reshape(-1), pos_table.reshape(-1))
    return (out5.transpose(2, 4, 0, 1, 3)
            .reshape(BATCH, SEQ_LEN, EMBED_DIM))
